# baseline (speedup 1.0000x reference)
import functools

import jax
import jax.numpy as jnp
from jax import lax
from jax.experimental import pallas as pl
from jax.experimental.pallas import tpu as pltpu
from jax.experimental.pallas import tpu_sc as plsc


def _build_sc_gather(B, D, n_workers, chunk):
    n_chunks_per_w = B // (n_workers * chunk)
    b_per_w = B // n_workers
    mesh = plsc.VectorSubcoreMesh(core_axis_name="c", subcore_axis_name="s")

    @functools.partial(
        pl.kernel,
        mesh=mesh,
        out_type=jax.ShapeDtypeStruct((B, D), jnp.float32),
        scratch_types=[
            pltpu.VMEM((chunk,), jnp.int32),
            pltpu.VMEM((chunk, D), jnp.float32),
            pltpu.SemaphoreType.DMA,
        ],
        compiler_params=pltpu.CompilerParams(use_tc_tiling_on_sc=False),
    )
    def sc_gather(idx_hbm, table_hbm, out_hbm, idx_v, rows_v, sem):
        num_cores = lax.axis_size("c")
        wid = lax.axis_index("s") * num_cores + lax.axis_index("c")
        base = wid * b_per_w

        def body(g, carry):
            off = pl.multiple_of(base + g * chunk, 8)
            pltpu.sync_copy(idx_hbm.at[pl.ds(off, chunk)], idx_v)
            pltpu.async_copy(table_hbm.at[idx_v], rows_v, sem).wait()
            pltpu.sync_copy(rows_v, out_hbm.at[pl.ds(off, chunk)])
            return carry

        lax.fori_loop(0, n_chunks_per_w, body, 0)

    return sc_gather


def kernel(tokens, embed_weights):
    S0, S1 = tokens.shape
    V, D = embed_weights.shape
    B = S0 * S1
    idx = tokens.reshape(B).astype(jnp.int32)
    n_workers = 32
    chunk = 1024
    out = _build_sc_gather(B, D, n_workers, chunk)(idx, embed_weights)
    return out.reshape(S0, S1, D)


# speedup vs baseline: 1.0450x; 1.0450x over previous
import functools

import jax
import jax.numpy as jnp
from jax import lax
from jax.experimental import pallas as pl
from jax.experimental.pallas import tpu as pltpu
from jax.experimental.pallas import tpu_sc as plsc


def _build_sc_gather(B, D, n_workers, chunk):
    b_per_w = B // n_workers
    n_chunks = b_per_w // chunk
    mesh = plsc.VectorSubcoreMesh(core_axis_name="c", subcore_axis_name="s")

    @functools.partial(
        pl.kernel,
        mesh=mesh,
        out_type=jax.ShapeDtypeStruct((B, D), jnp.float32),
        scratch_types=[
            pltpu.VMEM((b_per_w,), jnp.int32),
            pltpu.VMEM((2, chunk, D), jnp.float32),
            pltpu.SemaphoreType.DMA,
            pltpu.SemaphoreType.DMA((2,)),
            pltpu.SemaphoreType.DMA((2,)),
        ],
        compiler_params=pltpu.CompilerParams(use_tc_tiling_on_sc=False),
    )
    def sc_gather(idx_hbm, table_hbm, out_hbm, idx_v, rows_v, sem_i, sem_g, sem_w):
        num_cores = lax.axis_size("c")
        wid = lax.axis_index("s") * num_cores + lax.axis_index("c")
        base = pl.multiple_of(wid * b_per_w, 8)

        pltpu.async_copy(idx_hbm.at[pl.ds(base, b_per_w)], idx_v, sem_i).wait()

        def start_gather(g):
            buf = lax.rem(g, 2)
            pltpu.make_async_copy(
                table_hbm.at[idx_v.at[pl.ds(g * chunk, chunk)]],
                rows_v.at[buf],
                sem_g.at[buf],
            ).start()

        def wait_gather_start_write(g):
            buf = lax.rem(g, 2)
            off = pl.multiple_of(base + g * chunk, 8)
            pltpu.make_async_copy(
                table_hbm.at[idx_v.at[pl.ds(g * chunk, chunk)]],
                rows_v.at[buf],
                sem_g.at[buf],
            ).wait()
            pltpu.make_async_copy(
                rows_v.at[buf],
                out_hbm.at[pl.ds(off, chunk)],
                sem_w.at[buf],
            ).start()

        def wait_write(g):
            buf = lax.rem(g, 2)
            off = pl.multiple_of(base + g * chunk, 8)
            pltpu.make_async_copy(
                rows_v.at[buf],
                out_hbm.at[pl.ds(off, chunk)],
                sem_w.at[buf],
            ).wait()

        def body(g, carry):
            @pl.when(g >= 2)
            def _():
                wait_write(g)

            start_gather(g)

            @pl.when(g >= 1)
            def _():
                wait_gather_start_write(g - 1)

            return carry

        lax.fori_loop(0, n_chunks, body, 0)
        wait_gather_start_write(n_chunks - 1)
        wait_write(n_chunks - 2)
        wait_write(n_chunks - 1)

    return sc_gather


def kernel(tokens, embed_weights):
    S0, S1 = tokens.shape
    V, D = embed_weights.shape
    B = S0 * S1
    idx = tokens.T.reshape(B).astype(jnp.int32)
    n_workers = 32
    chunk = 512
    rows = _build_sc_gather(B, D, n_workers, chunk)(idx, embed_weights)
    return rows.reshape(S1, S0, D).transpose(1, 0, 2)
